# baseline (speedup 1.0000x reference)
import jax
import jax.numpy as jnp
from jax.experimental import pallas as pl
from jax.experimental.pallas import tpu as pltpu

_ROWS = 16384
_COLS = 64
_NC = 2
_CW = _ROWS // _NC
_NS = 4
_SW = _CW // _NS


def _mul_body(u_hbm, v_hbm, o_hbm, u_v, v_v, o_v, sem_u, sem_v, sem_o):
    cps_u = []
    cps_v = []
    for c in range(_NC):
        sl = pl.ds(c * _CW, _CW)
        cp_u = pltpu.make_async_copy(u_hbm.at[:, sl], u_v.at[:, sl], sem_u.at[c])
        cp_v = pltpu.make_async_copy(v_hbm.at[:, sl], v_v.at[:, sl], sem_v.at[c])
        cp_u.start()
        cp_v.start()
        cps_u.append(cp_u)
        cps_v.append(cp_v)
    cps_o = []
    for c in range(_NC):
        cps_u[c].wait()
        cps_v[c].wait()
        for s in range(_NS):
            k = c * _NS + s
            sl = pl.ds(k * _SW, _SW)
            o_v[:, sl] = u_v[:, sl] * v_v[:, sl]
            cp_o = pltpu.make_async_copy(o_v.at[:, sl], o_hbm.at[:, sl], sem_o.at[k])
            cp_o.start()
            cps_o.append(cp_o)
    for cp in cps_o:
        cp.wait()


@jax.jit
def kernel(user_emb, item_emb):
    u = user_emb.T
    v = item_emb.T
    any_spec = pl.BlockSpec(memory_space=pl.ANY)
    out = pl.pallas_call(
        _mul_body,
        in_specs=[any_spec, any_spec],
        out_specs=any_spec,
        out_shape=jax.ShapeDtypeStruct((_COLS, _ROWS), jnp.float32),
        scratch_shapes=[
            pltpu.VMEM((_COLS, _ROWS), jnp.float32),
            pltpu.VMEM((_COLS, _ROWS), jnp.float32),
            pltpu.VMEM((_COLS, _ROWS), jnp.float32),
            pltpu.SemaphoreType.DMA((_NC,)),
            pltpu.SemaphoreType.DMA((_NC,)),
            pltpu.SemaphoreType.DMA((_NC * _NS,)),
        ],
    )(u, v)
    return out.T


# speedup vs baseline: 1.0341x; 1.0341x over previous
import jax
import jax.numpy as jnp
from jax.experimental import pallas as pl
from jax.experimental.pallas import tpu as pltpu

_ROWS = 16384
_COLS = 64
_NC = 2
_CW = _ROWS // _NC
_NS = 2
_SW = _CW // _NS


def _mul_body(u_hbm, v_hbm, o_hbm, u_v, v_v, o_v, sem_u, sem_v, sem_o):
    cps_u = []
    cps_v = []
    for c in range(_NC):
        sl = pl.ds(c * _CW, _CW)
        cp_u = pltpu.make_async_copy(u_hbm.at[:, sl], u_v.at[:, sl], sem_u.at[c])
        cp_v = pltpu.make_async_copy(v_hbm.at[:, sl], v_v.at[:, sl], sem_v.at[c])
        cp_u.start()
        cp_v.start()
        cps_u.append(cp_u)
        cps_v.append(cp_v)
    cps_o = []
    for c in range(_NC):
        cps_u[c].wait()
        cps_v[c].wait()
        for s in range(_NS):
            k = c * _NS + s
            sl = pl.ds(k * _SW, _SW)
            o_v[:, sl] = u_v[:, sl] * v_v[:, sl]
            cp_o = pltpu.make_async_copy(o_v.at[:, sl], o_hbm.at[:, sl], sem_o.at[k])
            cp_o.start()
            cps_o.append(cp_o)
    for cp in cps_o:
        cp.wait()


@jax.jit
def kernel(user_emb, item_emb):
    u = user_emb.T
    v = item_emb.T
    any_spec = pl.BlockSpec(memory_space=pl.ANY)
    out = pl.pallas_call(
        _mul_body,
        in_specs=[any_spec, any_spec],
        out_specs=any_spec,
        out_shape=jax.ShapeDtypeStruct((_COLS, _ROWS), jnp.float32),
        scratch_shapes=[
            pltpu.VMEM((_COLS, _ROWS), jnp.float32),
            pltpu.VMEM((_COLS, _ROWS), jnp.float32),
            pltpu.VMEM((_COLS, _ROWS), jnp.float32),
            pltpu.SemaphoreType.DMA((_NC,)),
            pltpu.SemaphoreType.DMA((_NC,)),
            pltpu.SemaphoreType.DMA((_NC * _NS,)),
        ],
    )(u, v)
    return out.T
